# baseline (speedup 1.0000x reference)
import functools

import jax
import jax.numpy as jnp
from jax import lax
from jax.experimental import pallas as pl
from jax.experimental.pallas import tpu as pltpu
from jax.experimental.pallas import tpu_sc as plsc

D_MODEL = 768
B_TOTAL = 4 * 8192
NUM_CORES = 2
NUM_SUBCORES = 16
NUM_WORKERS = NUM_CORES * NUM_SUBCORES
B_PER_WORKER = B_TOTAL // NUM_WORKERS
W_PER_ROW = NUM_WORKERS // 4
NQ = 16
STEP = 8

_mesh = plsc.VectorSubcoreMesh(core_axis_name="c", subcore_axis_name="s")


@jax.jit
def _sc_gather(pe, position_ids):
    @functools.partial(
        pl.kernel,
        mesh=_mesh,
        out_type=jax.ShapeDtypeStruct((B_TOTAL, D_MODEL), jnp.float32),
        scratch_types=[
            pltpu.SMEM((B_PER_WORKER,), jnp.int32),
            pltpu.VMEM_SHARED((NUM_SUBCORES, B_PER_WORKER), jnp.int32),
            pltpu.SemaphoreType.DMA,
        ],
    )
    def k(table_hbm, idx_hbm, out_hbm, idx_s, idx_sh, sem):
        wid = lax.axis_index("s") * NUM_CORES + lax.axis_index("c")
        sid = lax.axis_index("s")
        base = wid * B_PER_WORKER
        pltpu.sync_copy(
            idx_hbm.at[wid // W_PER_ROW,
                       pl.ds((wid % W_PER_ROW) * B_PER_WORKER, B_PER_WORKER)],
            idx_sh.at[sid],
        )
        pltpu.sync_copy(idx_sh.at[sid], idx_s)

        def copy_row(i):
            pltpu.async_copy(
                table_hbm.at[pl.ds(idx_s[i], 1)],
                out_hbm.at[pl.ds(base + i, 1)],
                sem,
            )

        def drain_rows(n):
            pltpu.make_async_copy(
                table_hbm.at[pl.ds(0, n)], out_hbm.at[pl.ds(base, n)], sem
            ).wait()

        for q in range(NQ):
            copy_row(q)

        @pl.loop(NQ, B_PER_WORKER, step=STEP)
        def _(i):
            drain_rows(STEP)
            for q in range(STEP):
                copy_row(i + q)

        drain_rows(STEP)
        drain_rows(STEP)

    return k(pe, position_ids)


def kernel(position_ids, pe):
    out = _sc_gather(pe, position_ids.astype(jnp.int32))
    return out.reshape(position_ids.shape + (pe.shape[1],))


# speedup vs baseline: 33.1977x; 33.1977x over previous
import functools

import jax
import jax.numpy as jnp
from jax import lax
from jax.experimental import pallas as pl
from jax.experimental.pallas import tpu as pltpu
from jax.experimental.pallas import tpu_sc as plsc

D_MODEL = 768
B_TOTAL = 4 * 8192
NUM_CORES = 2
NUM_SUBCORES = 16
NUM_WORKERS = NUM_CORES * NUM_SUBCORES
B_PER_WORKER = B_TOTAL // NUM_WORKERS
W_PER_ROW = NUM_WORKERS // 4
CHUNK = 64
NUM_CHUNKS = B_PER_WORKER // CHUNK

_mesh = plsc.VectorSubcoreMesh(core_axis_name="c", subcore_axis_name="s")


@jax.jit
def _sc_gather(pe, position_ids):
    @functools.partial(
        pl.kernel,
        mesh=_mesh,
        out_type=jax.ShapeDtypeStruct((B_TOTAL, D_MODEL), jnp.float32),
        scratch_types=[
            pltpu.VMEM((B_PER_WORKER,), jnp.int32),
            pltpu.VMEM((2, CHUNK, D_MODEL), jnp.float32),
            pltpu.SemaphoreType.DMA((2,)),
            pltpu.SemaphoreType.DMA((2,)),
        ],
    )
    def k(table_hbm, idx_hbm, out_hbm, idx_v, rows_v, gsem, ssem):
        wid = lax.axis_index("s") * NUM_CORES + lax.axis_index("c")
        base = wid * B_PER_WORKER
        pltpu.sync_copy(
            idx_hbm.at[wid // W_PER_ROW,
                       pl.ds((wid % W_PER_ROW) * B_PER_WORKER, B_PER_WORKER)],
            idx_v,
        )

        HALF = CHUNK // 2

        def start_gather(b, c):
            h0 = pltpu.async_copy(
                table_hbm.at[idx_v.at[pl.ds(c * CHUNK, HALF)]],
                rows_v.at[b, pl.ds(0, HALF)], gsem.at[b],
            )
            h1 = pltpu.async_copy(
                table_hbm.at[idx_v.at[pl.ds(c * CHUNK + HALF, HALF)]],
                rows_v.at[b, pl.ds(HALF, HALF)], gsem.at[b],
            )
            return (h0, h1)

        def start_store(b, c):
            return pltpu.async_copy(
                rows_v.at[b], out_hbm.at[pl.ds(base + c * CHUNK, CHUNK)],
                ssem.at[b],
            )

        g = [None, None]
        s = [None, None]
        g[0] = start_gather(0, 0)
        for c in range(NUM_CHUNKS):
            b = c & 1
            nb = 1 - b
            if c + 1 < NUM_CHUNKS:
                if s[nb] is not None:
                    s[nb].wait()
                g[nb] = start_gather(nb, c + 1)
            g[b][0].wait()
            g[b][1].wait()
            s[b] = start_store(b, c)
        s[0].wait()
        s[1].wait()

    return k(pe, position_ids)


def kernel(position_ids, pe):
    out = _sc_gather(pe, position_ids.astype(jnp.int32))
    return out.reshape(position_ids.shape + (pe.shape[1],))
